# baseline (speedup 1.0000x reference)
import functools
import jax
import jax.numpy as jnp
from jax import lax
from jax.experimental import pallas as pl
from jax.experimental.pallas import tpu as pltpu
from jax.experimental.pallas import tpu_sc as plsc

_NC, _NS, _L = 2, 16, 16
_NW = _NC * _NS


def _sc_body(num_patches, dim, batch, ppw, cp,
             x_hbm, table_hbm, pos_hbm, out_hbm,
             idx_v, tbuf, xbuf, in_sems, out_sem, gat_sem):
    wid = lax.axis_index("s") * _NC + lax.axis_index("c")
    base = wid * ppw
    pltpu.sync_copy(pos_hbm.at[pl.ds(base, ppw)], idx_v)

    n_chunks = ppw // cp
    for c in range(n_chunks):
        idx_slice = idx_v.at[pl.ds(c * cp, cp)]
        pltpu.async_copy(table_hbm.at[idx_slice], tbuf, gat_sem).wait()

        in_handles = []
        for b in range(batch):
            h = pltpu.async_copy(
                x_hbm.at[b, pl.ds(base + c * cp, cp)], xbuf.at[b],
                in_sems.at[b])
            in_handles.append(h)

        out_handles = []
        for b in range(batch):
            in_handles[b].wait()

            def row_add(r, _, b=b):
                for k in range(dim // _L):
                    sl = pl.ds(k * _L, _L)
                    plsc.addupdate(xbuf.at[b, r, sl], tbuf[r, sl])
                return _

            lax.fori_loop(0, cp, row_add, None)
            out_handles.append(pltpu.async_copy(
                xbuf.at[b], out_hbm.at[b, pl.ds(base + c * cp, cp)],
                out_sem))
        for h in out_handles:
            h.wait()


def sc_kernel(encoded_patches, position_embedding, positions):
    batch, num_patches, dim = encoded_patches.shape
    ppw = num_patches // _NW
    cp = 16

    mesh = plsc.VectorSubcoreMesh(core_axis_name="c", subcore_axis_name="s")
    body = functools.partial(_sc_body, num_patches, dim, batch, ppw, cp)
    return pl.kernel(
        body,
        out_type=jax.ShapeDtypeStruct(encoded_patches.shape, encoded_patches.dtype),
        mesh=mesh,
        scratch_types=[
            pltpu.VMEM((ppw,), jnp.int32),
            pltpu.VMEM((cp, dim), jnp.float32),
            pltpu.VMEM((batch, cp, dim), jnp.float32),
            pltpu.SemaphoreType.DMA((batch,)),
            pltpu.SemaphoreType.DMA,
            pltpu.SemaphoreType.DMA,
        ],
    )(encoded_patches, position_embedding, positions)


def kernel(encoded_patches, position_embedding, positions):
    return sc_kernel(encoded_patches, position_embedding, positions)
